# baseline (speedup 1.0000x reference)
import functools

import jax
import jax.numpy as jnp
from jax import lax
from jax.experimental import pallas as pl
from jax.experimental.pallas import tpu as pltpu
from jax.experimental.pallas import tpu_sc as plsc

N_NODES = 10000
N_EDGES = 320000
D_IN = 128
H1, C1 = 8, 8
H2, C2 = 1, 40

NC, NS = 2, 16
NW = NC * NS
K_CHUNK = 112
NBUF = 5
DEPTH = 4
N_PAD = 10112
E_PAD = ((N_EDGES + NW * K_CHUNK * NBUF - 1)
         // (NW * K_CHUNK * NBUF)) * NW * K_CHUNK * NBUF
M0 = 100
M1 = (E_PAD // K_CHUNK - 16 * M0) // 16
MMAX = max(M0, M1)
E_ALLOC_ROWS = 16 * M0 + 15 * M1 + MMAX


def _edge_pass(Wt, H, C, name):
    AOFF = H * C
    EW = E_PAD // NW
    n_chunks = EW // K_CHUNK
    rows_per_s = N_PAD // NS
    mesh = plsc.VectorSubcoreMesh(core_axis_name="c", subcore_axis_name="s")

    @functools.partial(
        pl.kernel,
        out_type=jax.ShapeDtypeStruct((NC, N_PAD, Wt), jnp.float32),
        mesh=mesh,
        scratch_types=[
            pltpu.VMEM((MMAX, K_CHUNK), jnp.int32),
            pltpu.VMEM((MMAX, K_CHUNK), jnp.int32),
            pltpu.VMEM((NBUF, K_CHUNK, Wt), jnp.float32),
            pltpu.VMEM((NBUF, K_CHUNK, 16), jnp.float32),
            pltpu.VMEM_SHARED((N_PAD, Wt), jnp.float32),
            pltpu.SemaphoreType.DMA((NBUF,)),
            pltpu.SemaphoreType.DMA((NBUF,)),
        ],
        compiler_params=pltpu.CompilerParams(use_tc_tiling_on_sc=False),
        name=name,
    )
    def kern(T_hbm, adst_hbm, src_hbm, dst_hbm, zeros_hbm, out_hbm,
             idxs_v, idxd_v, rows_v, adr_v, acc_sh, sem_g, sem_s):
        c = lax.axis_index("c")
        s = lax.axis_index("s")
        m = jnp.where(c == 0, M0, M1)
        start = jnp.where(c == 0, s * M0, 16 * M0 + s * M1)
        pltpu.sync_copy(zeros_hbm.at[pl.ds(s * rows_per_s, rows_per_s)],
                        acc_sh.at[pl.ds(s * rows_per_s, rows_per_s)])
        pltpu.sync_copy(src_hbm.at[pl.ds(start, MMAX)], idxs_v)
        pltpu.sync_copy(dst_hbm.at[pl.ds(start, MMAX)], idxd_v)
        plsc.subcore_barrier()

        iota = lax.iota(jnp.int32, 16)
        dnums = lax.GatherDimensionNumbers(
            offset_dims=(), collapsed_slice_dims=(0,), start_index_map=(0,))

        def vgather(v, idx):
            return lax.gather(v, idx[:, None], dnums, slice_sizes=(1,),
                              mode=lax.GatherScatterMode.PROMISE_IN_BOUNDS)

        def fetch(ch, b):
            pltpu.async_copy(T_hbm.at[idxs_v.at[ch]], rows_v.at[b],
                             sem_g.at[b])
            pltpu.async_copy(adst_hbm.at[idxd_v.at[ch]], adr_v.at[b],
                             sem_g.at[b])

        def wait_gather(ch, b):
            pltpu.make_async_copy(T_hbm.at[idxs_v.at[ch]], rows_v.at[b],
                                  sem_g.at[b]).wait()
            pltpu.make_async_copy(adst_hbm.at[idxd_v.at[ch]], adr_v.at[b],
                                  sem_g.at[b]).wait()

        def wait_scatter(ch, b):
            pltpu.make_async_copy(rows_v.at[b], acc_sh.at[idxd_v.at[ch]],
                                  sem_s.at[b]).wait()

        def compute(b):
            @plsc.parallel_loop(0, K_CHUNK, step=1, unroll=8)
            def row_body(k):
                adr = adr_v[b, k, :]
                if H == 8:
                    va = rows_v[b, k, pl.ds(AOFF, 16)]
                    a = va + adr
                    a = jnp.where(a > 0, a, 0.2 * a)
                    ew = jnp.exp(a)
                    rows_v[b, k, pl.ds(AOFF, 16)] = ew
                    for j in range(AOFF // 16):
                        rep = vgather(ew, 2 * j + (iota >> 3))
                        rows_v[b, k, pl.ds(16 * j, 16)] = (
                            rows_v[b, k, pl.ds(16 * j, 16)] * rep)
                else:
                    v2 = rows_v[b, k, pl.ds(32, 16)]
                    a = vgather(v2, jnp.full((16,), 8, jnp.int32)) + \
                        vgather(adr, jnp.zeros((16,), jnp.int32))
                    a = jnp.where(a > 0, a, 0.2 * a)
                    ew = jnp.exp(a)
                    for j in range(2):
                        rows_v[b, k, pl.ds(16 * j, 16)] = (
                            rows_v[b, k, pl.ds(16 * j, 16)] * ew)
                    rows_v[b, k, pl.ds(32, 16)] = jnp.where(
                        iota < 8, v2 * ew, ew)

        for b in range(DEPTH):
            fetch(b, b)

        def outer_body(blk, carry):
            for b in range(NBUF):
                ch = blk * NBUF + b
                wait_gather(ch, b)
                compute(b)
                pltpu.async_copy(rows_v.at[b], acc_sh.at[idxd_v.at[ch]],
                                 sem_s.at[b], add=True)
                ch2 = ch + DEPTH
                bp = (b + DEPTH) % NBUF

                @pl.when(ch2 < m)
                def _():
                    @pl.when(ch2 >= NBUF)
                    def _():
                        wait_scatter(ch2 - NBUF, bp)
                    fetch(ch2, bp)
            return carry

        lax.fori_loop(0, m // NBUF, outer_body, 0)
        for b in range(NBUF):
            wait_scatter(m - NBUF + b, b)
        plsc.subcore_barrier()
        pltpu.sync_copy(acc_sh.at[pl.ds(s * rows_per_s, rows_per_s)],
                        out_hbm.at[c, pl.ds(s * rows_per_s, rows_per_s)])

    return kern


def _table1_body(x_ref, w_ref, t_ref, adst_ref):
    t = jnp.dot(x_ref[...], w_ref[...], preferred_element_type=jnp.float32)
    t_ref[...] = t
    adst_ref[...] = jnp.concatenate(
        [t[:, H1 * C1 + H1:H1 * C1 + 2 * H1],
         jnp.zeros((t.shape[0], 8), jnp.float32)], axis=1)


def _mid_body(acc_ref, t1_ref, b1_ref, w2_ref, r1_ref, t2_ref, adst2_ref):
    HC = H1 * C1
    num = acc_ref[0, :, :HC] + acc_ref[1, :, :HC]
    den = acc_ref[0, :, HC:HC + H1] + acc_ref[1, :, HC:HC + H1]
    t1 = t1_ref[...]
    xp = t1[:, :HC]
    a = t1[:, HC:HC + H1] + t1[:, HC + H1:HC + 2 * H1]
    a = jnp.where(a > 0, a, 0.2 * a)
    ews = jnp.exp(a)
    r1 = r1_ref[...]
    num = num + xp * jnp.dot(ews, r1, preferred_element_type=jnp.float32)
    den = jnp.dot(den + ews, r1, preferred_element_type=jnp.float32)
    h = num / (den + 1e-16) + b1_ref[...]
    h = jnp.where(h > 0, h, jnp.exp(h) - 1.0)
    t2 = jnp.dot(h, w2_ref[...], preferred_element_type=jnp.float32)
    t2_ref[...] = t2
    adst2_ref[...] = jnp.concatenate(
        [t2[:, 41:42], jnp.zeros((t2.shape[0], 15), jnp.float32)], axis=1)


def _final_body(acc_ref, t2_ref, b2_ref, f_ref, lsm_ref):
    HC = H2 * C2
    num = acc_ref[0, :, :HC] + acc_ref[1, :, :HC]
    den = acc_ref[0, :, HC:HC + 1] + acc_ref[1, :, HC:HC + 1]
    t2 = t2_ref[...]
    xp = t2[:, :HC]
    a = t2[:, HC:HC + 1] + t2[:, HC + 1:HC + 2]
    a = jnp.where(a > 0, a, 0.2 * a)
    ews = jnp.exp(a)
    f = (num + xp * ews) / (den + ews + 1e-16) + b2_ref[...]
    f_ref[...] = f
    m = jnp.max(f, axis=1, keepdims=True)
    lsm_ref[...] = f - m - jnp.log(jnp.sum(jnp.exp(f - m), axis=1, keepdims=True))


_edge1 = _edge_pass(80, H1, C1, "gat_edge1")
_edge2 = _edge_pass(48, H2, C2, "gat_edge2")


def kernel(x, edge_index, W1, att_src1, att_dst1, b1, W2, att_src2, att_dst2, b2):
    f32 = jnp.float32
    def blockdiag(att, H, C):
        r = jnp.arange(H * C)
        return jnp.zeros((H * C, H), f32).at[r, r // C].set(att.reshape(-1))

    W1p = jnp.concatenate(
        [W1, W1 @ blockdiag(att_src1, H1, C1), W1 @ blockdiag(att_dst1, H1, C1)],
        axis=1)
    W2p = jnp.concatenate(
        [W2, W2 @ blockdiag(att_src2, H2, C2), W2 @ blockdiag(att_dst2, H2, C2)],
        axis=1)
    W2p = jnp.pad(W2p, ((0, 0), (0, 48 - W2p.shape[1])))
    R1 = jnp.repeat(jnp.eye(H1, dtype=f32), C1, axis=1)

    xpad = jnp.pad(x, ((0, N_PAD - N_NODES), (0, 0)))
    e_alloc = E_ALLOC_ROWS * K_CHUNK
    src = jnp.pad(edge_index[0].astype(jnp.int32), (0, e_alloc - N_EDGES),
                  constant_values=N_NODES).reshape(-1, K_CHUNK)
    dst = jnp.pad(edge_index[1].astype(jnp.int32), (0, e_alloc - N_EDGES),
                  constant_values=N_NODES).reshape(-1, K_CHUNK)
    zeros80 = jnp.zeros((N_PAD, 80), f32)
    zeros48 = jnp.zeros((N_PAD, 48), f32)

    T1, adst1 = pl.pallas_call(
        _table1_body,
        out_shape=(jax.ShapeDtypeStruct((N_PAD, 80), f32),
                   jax.ShapeDtypeStruct((N_PAD, 16), f32)),
    )(xpad, W1p)

    acc1 = _edge1(T1, adst1, src, dst, zeros80)

    T2, adst2 = pl.pallas_call(
        _mid_body,
        out_shape=(jax.ShapeDtypeStruct((N_PAD, 48), f32),
                   jax.ShapeDtypeStruct((N_PAD, 16), f32)),
    )(acc1, T1, b1.reshape(1, -1), W2p, R1)

    acc2 = _edge2(T2, adst2, src, dst, zeros48)

    f, lsm = pl.pallas_call(
        _final_body,
        out_shape=(jax.ShapeDtypeStruct((N_PAD, C2), f32),
                   jax.ShapeDtypeStruct((N_PAD, C2), f32)),
    )(acc2, T2, b2.reshape(1, -1))

    return (f[:N_NODES], lsm[:N_NODES])


# speedup vs baseline: 1.0162x; 1.0162x over previous
import functools

import jax
import jax.numpy as jnp
from jax import lax
from jax.experimental import pallas as pl
from jax.experimental.pallas import tpu as pltpu
from jax.experimental.pallas import tpu_sc as plsc

N_NODES = 10000
N_EDGES = 320000
D_IN = 128
H1, C1 = 8, 8
H2, C2 = 1, 40

NC, NS = 2, 16
NW = NC * NS
K_CHUNK = 112
NBUF = 5
DEPTH = 4
N_PAD = 10112
E_PAD = ((N_EDGES + NW * K_CHUNK * NBUF - 1)
         // (NW * K_CHUNK * NBUF)) * NW * K_CHUNK * NBUF
M0 = 115
M1 = (E_PAD // K_CHUNK - 16 * M0) // 16
MMAX = max(M0, M1)
E_ALLOC_ROWS = 16 * M0 + 15 * M1 + MMAX


def _edge_pass(Wt, H, C, name):
    AOFF = H * C
    EW = E_PAD // NW
    n_chunks = EW // K_CHUNK
    rows_per_s = N_PAD // NS
    mesh = plsc.VectorSubcoreMesh(core_axis_name="c", subcore_axis_name="s")

    @functools.partial(
        pl.kernel,
        out_type=jax.ShapeDtypeStruct((NC, N_PAD, Wt), jnp.float32),
        mesh=mesh,
        scratch_types=[
            pltpu.VMEM((MMAX, K_CHUNK), jnp.int32),
            pltpu.VMEM((MMAX, K_CHUNK), jnp.int32),
            pltpu.VMEM((NBUF, K_CHUNK, Wt), jnp.float32),
            pltpu.VMEM((NBUF, K_CHUNK, 16), jnp.float32),
            pltpu.VMEM_SHARED((N_PAD, Wt), jnp.float32),
            pltpu.SemaphoreType.DMA((NBUF,)),
            pltpu.SemaphoreType.DMA((NBUF,)),
        ],
        compiler_params=pltpu.CompilerParams(use_tc_tiling_on_sc=False),
        name=name,
    )
    def kern(T_hbm, adst_hbm, src_hbm, dst_hbm, zeros_hbm, out_hbm,
             idxs_v, idxd_v, rows_v, adr_v, acc_sh, sem_g, sem_s):
        c = lax.axis_index("c")
        s = lax.axis_index("s")
        m = jnp.where(c == 0, M0, M1)
        start = jnp.where(c == 0, s * M0, 16 * M0 + s * M1)
        pltpu.sync_copy(zeros_hbm.at[pl.ds(s * rows_per_s, rows_per_s)],
                        acc_sh.at[pl.ds(s * rows_per_s, rows_per_s)])
        pltpu.sync_copy(src_hbm.at[pl.ds(start, MMAX)], idxs_v)
        pltpu.sync_copy(dst_hbm.at[pl.ds(start, MMAX)], idxd_v)
        plsc.subcore_barrier()

        iota = lax.iota(jnp.int32, 16)
        dnums = lax.GatherDimensionNumbers(
            offset_dims=(), collapsed_slice_dims=(0,), start_index_map=(0,))

        def vgather(v, idx):
            return lax.gather(v, idx[:, None], dnums, slice_sizes=(1,),
                              mode=lax.GatherScatterMode.PROMISE_IN_BOUNDS)

        def fetch(ch, b):
            pltpu.async_copy(T_hbm.at[idxs_v.at[ch]], rows_v.at[b],
                             sem_g.at[b])
            pltpu.async_copy(adst_hbm.at[idxd_v.at[ch]], adr_v.at[b],
                             sem_g.at[b])

        def wait_gather(ch, b):
            pltpu.make_async_copy(T_hbm.at[idxs_v.at[ch]], rows_v.at[b],
                                  sem_g.at[b]).wait()
            pltpu.make_async_copy(adst_hbm.at[idxd_v.at[ch]], adr_v.at[b],
                                  sem_g.at[b]).wait()

        def wait_scatter(ch, b):
            pltpu.make_async_copy(rows_v.at[b], acc_sh.at[idxd_v.at[ch]],
                                  sem_s.at[b]).wait()

        def compute(b):
            @plsc.parallel_loop(0, K_CHUNK, step=1, unroll=8)
            def row_body(k):
                adr = adr_v[b, k, :]
                if H == 8:
                    va = rows_v[b, k, pl.ds(AOFF, 16)]
                    a = va + adr
                    a = jnp.where(a > 0, a, 0.2 * a)
                    ew = jnp.exp(a)
                    rows_v[b, k, pl.ds(AOFF, 16)] = ew
                    for j in range(AOFF // 16):
                        rep = vgather(ew, 2 * j + (iota >> 3))
                        rows_v[b, k, pl.ds(16 * j, 16)] = (
                            rows_v[b, k, pl.ds(16 * j, 16)] * rep)
                else:
                    v2 = rows_v[b, k, pl.ds(32, 16)]
                    a = vgather(v2, jnp.full((16,), 8, jnp.int32)) + \
                        vgather(adr, jnp.zeros((16,), jnp.int32))
                    a = jnp.where(a > 0, a, 0.2 * a)
                    ew = jnp.exp(a)
                    for j in range(2):
                        rows_v[b, k, pl.ds(16 * j, 16)] = (
                            rows_v[b, k, pl.ds(16 * j, 16)] * ew)
                    rows_v[b, k, pl.ds(32, 16)] = jnp.where(
                        iota < 8, v2 * ew, ew)

        for b in range(DEPTH):
            fetch(b, b)

        def outer_body(blk, carry):
            for b in range(NBUF):
                ch = blk * NBUF + b
                wait_gather(ch, b)
                compute(b)
                pltpu.async_copy(rows_v.at[b], acc_sh.at[idxd_v.at[ch]],
                                 sem_s.at[b], add=True)
                ch2 = ch + DEPTH
                bp = (b + DEPTH) % NBUF

                @pl.when(ch2 < m)
                def _():
                    @pl.when(ch2 >= NBUF)
                    def _():
                        wait_scatter(ch2 - NBUF, bp)
                    fetch(ch2, bp)
            return carry

        lax.fori_loop(0, m // NBUF, outer_body, 0)
        for b in range(NBUF):
            wait_scatter(m - NBUF + b, b)
        plsc.subcore_barrier()
        pltpu.sync_copy(acc_sh.at[pl.ds(s * rows_per_s, rows_per_s)],
                        out_hbm.at[c, pl.ds(s * rows_per_s, rows_per_s)])

    return kern


def _table1_body(x_ref, w_ref, t_ref, adst_ref):
    t = jnp.dot(x_ref[...], w_ref[...], preferred_element_type=jnp.float32)
    t_ref[...] = t
    adst_ref[...] = jnp.concatenate(
        [t[:, H1 * C1 + H1:H1 * C1 + 2 * H1],
         jnp.zeros((t.shape[0], 8), jnp.float32)], axis=1)


def _mid_body(acc_ref, t1_ref, b1_ref, w2_ref, r1_ref, t2_ref, adst2_ref):
    HC = H1 * C1
    num = acc_ref[0, :, :HC] + acc_ref[1, :, :HC]
    den = acc_ref[0, :, HC:HC + H1] + acc_ref[1, :, HC:HC + H1]
    t1 = t1_ref[...]
    xp = t1[:, :HC]
    a = t1[:, HC:HC + H1] + t1[:, HC + H1:HC + 2 * H1]
    a = jnp.where(a > 0, a, 0.2 * a)
    ews = jnp.exp(a)
    r1 = r1_ref[...]
    num = num + xp * jnp.dot(ews, r1, preferred_element_type=jnp.float32)
    den = jnp.dot(den + ews, r1, preferred_element_type=jnp.float32)
    h = num / (den + 1e-16) + b1_ref[...]
    h = jnp.where(h > 0, h, jnp.exp(h) - 1.0)
    t2 = jnp.dot(h, w2_ref[...], preferred_element_type=jnp.float32)
    t2_ref[...] = t2
    adst2_ref[...] = jnp.concatenate(
        [t2[:, 41:42], jnp.zeros((t2.shape[0], 15), jnp.float32)], axis=1)


def _final_body(acc_ref, t2_ref, b2_ref, f_ref, lsm_ref):
    HC = H2 * C2
    num = acc_ref[0, :, :HC] + acc_ref[1, :, :HC]
    den = acc_ref[0, :, HC:HC + 1] + acc_ref[1, :, HC:HC + 1]
    t2 = t2_ref[...]
    xp = t2[:, :HC]
    a = t2[:, HC:HC + 1] + t2[:, HC + 1:HC + 2]
    a = jnp.where(a > 0, a, 0.2 * a)
    ews = jnp.exp(a)
    f = (num + xp * ews) / (den + ews + 1e-16) + b2_ref[...]
    f_ref[...] = f
    m = jnp.max(f, axis=1, keepdims=True)
    lsm_ref[...] = f - m - jnp.log(jnp.sum(jnp.exp(f - m), axis=1, keepdims=True))


_edge1 = _edge_pass(80, H1, C1, "gat_edge1")
_edge2 = _edge_pass(48, H2, C2, "gat_edge2")


def kernel(x, edge_index, W1, att_src1, att_dst1, b1, W2, att_src2, att_dst2, b2):
    f32 = jnp.float32
    def blockdiag(att, H, C):
        r = jnp.arange(H * C)
        return jnp.zeros((H * C, H), f32).at[r, r // C].set(att.reshape(-1))

    W1p = jnp.concatenate(
        [W1, W1 @ blockdiag(att_src1, H1, C1), W1 @ blockdiag(att_dst1, H1, C1)],
        axis=1)
    W2p = jnp.concatenate(
        [W2, W2 @ blockdiag(att_src2, H2, C2), W2 @ blockdiag(att_dst2, H2, C2)],
        axis=1)
    W2p = jnp.pad(W2p, ((0, 0), (0, 48 - W2p.shape[1])))
    R1 = jnp.repeat(jnp.eye(H1, dtype=f32), C1, axis=1)

    xpad = jnp.pad(x, ((0, N_PAD - N_NODES), (0, 0)))
    e_alloc = E_ALLOC_ROWS * K_CHUNK
    src = jnp.pad(edge_index[0].astype(jnp.int32), (0, e_alloc - N_EDGES),
                  constant_values=N_NODES).reshape(-1, K_CHUNK)
    dst = jnp.pad(edge_index[1].astype(jnp.int32), (0, e_alloc - N_EDGES),
                  constant_values=N_NODES).reshape(-1, K_CHUNK)
    zeros80 = jnp.zeros((N_PAD, 80), f32)
    zeros48 = jnp.zeros((N_PAD, 48), f32)

    T1, adst1 = pl.pallas_call(
        _table1_body,
        out_shape=(jax.ShapeDtypeStruct((N_PAD, 80), f32),
                   jax.ShapeDtypeStruct((N_PAD, 16), f32)),
    )(xpad, W1p)

    acc1 = _edge1(T1, adst1, src, dst, zeros80)

    T2, adst2 = pl.pallas_call(
        _mid_body,
        out_shape=(jax.ShapeDtypeStruct((N_PAD, 48), f32),
                   jax.ShapeDtypeStruct((N_PAD, 16), f32)),
    )(acc1, T1, b1.reshape(1, -1), W2p, R1)

    acc2 = _edge2(T2, adst2, src, dst, zeros48)

    f, lsm = pl.pallas_call(
        _final_body,
        out_shape=(jax.ShapeDtypeStruct((N_PAD, C2), f32),
                   jax.ShapeDtypeStruct((N_PAD, C2), f32)),
    )(acc2, T2, b2.reshape(1, -1))

    return (f[:N_NODES], lsm[:N_NODES])
